# Initial kernel scaffold; baseline (speedup 1.0000x reference)
import jax
import jax.numpy as jnp
from jax.experimental import pallas as pl


def kernel(z, edge_index):
    raise NotImplementedError("write your pallas kernel here")


# speedup vs baseline: 2.5498x; 2.5498x over previous
import functools

import jax
import jax.numpy as jnp
from jax import lax
from jax.experimental import pallas as pl
from jax.experimental.pallas import tpu as pltpu
from jax.experimental.pallas import tpu_sc as plsc

_NC = 2
_NS = 16
_NW = _NC * _NS
_CHUNK = 80

_DNUMS = lax.GatherDimensionNumbers(
    offset_dims=(), collapsed_slice_dims=(0,), start_index_map=(0,))


def _take16(x, idx):
    return lax.gather(x, idx[:, None], _DNUMS, (1,),
                      mode=lax.GatherScatterMode.PROMISE_IN_BOUNDS)


def _build(n_nodes, d_feat, n_edges):
    assert n_edges % _NW == 0
    edges_per_w = n_edges // _NW
    assert edges_per_w % _CHUNK == 0
    n_chunks = edges_per_w // _CHUNK
    n_grp = _CHUNK // 16
    n_k = d_feat // 16

    mesh = plsc.VectorSubcoreMesh(
        core_axis_name="c", subcore_axis_name="s",
        num_cores=_NC, num_subcores=_NS)

    @functools.partial(
        pl.kernel,
        out_type=jax.ShapeDtypeStruct((n_edges,), jnp.float32),
        mesh=mesh,
        scratch_types=[
            pltpu.VMEM((_CHUNK,), jnp.int32),
            pltpu.VMEM((_CHUNK,), jnp.int32),
            pltpu.VMEM((_CHUNK, d_feat), jnp.float32),
            pltpu.VMEM((_CHUNK, d_feat), jnp.float32),
            pltpu.VMEM((_CHUNK,), jnp.float32),
            pltpu.SemaphoreType.DMA,
            pltpu.SemaphoreType.DMA,
        ],
    )
    def decoder(z_hbm, src_hbm, dst_hbm, out_hbm, si_v, di_v, sr_v, dr_v, o_v,
                sem_s, sem_d):
        wid = lax.axis_index("s") * _NC + lax.axis_index("c")
        base_w = wid * edges_per_w
        lane = lax.iota(jnp.int32, 16)
        masks = [(lane & d) == 0 for d in (1, 2, 4, 8)]
        perms = [lane ^ d for d in (1, 2, 4, 8)]

        @pl.loop(0, n_chunks)
        def _chunk(i):
            base = base_w + i * _CHUNK
            pltpu.sync_copy(src_hbm.at[pl.ds(base, _CHUNK)], si_v)
            pltpu.sync_copy(dst_hbm.at[pl.ds(base, _CHUNK)], di_v)
            cp_s = pltpu.async_copy(z_hbm.at[si_v], sr_v, sem_s)
            cp_d = pltpu.async_copy(z_hbm.at[di_v], dr_v, sem_d)
            cp_s.wait()
            cp_d.wait()

            @pl.loop(0, n_grp)
            def _grp(g):
                vecs = []
                for e in range(16):
                    r = g * 16 + e
                    acc = sr_v[r, pl.ds(0, 16)] * dr_v[r, pl.ds(0, 16)]
                    for k in range(1, n_k):
                        acc = acc + (sr_v[r, pl.ds(k * 16, 16)]
                                     * dr_v[r, pl.ds(k * 16, 16)])
                    vecs.append(acc)
                for m, p in zip(masks, perms):
                    vecs = [
                        jnp.where(m, a, _take16(b, p))
                        + jnp.where(m, _take16(a, p), b)
                        for a, b in zip(vecs[0::2], vecs[1::2])
                    ]
                res = vecs[0]
                o_v[pl.ds(g * 16, 16)] = 1.0 / (1.0 + jnp.exp(-res))

            pltpu.sync_copy(o_v, out_hbm.at[pl.ds(base, _CHUNK)])

    return decoder


def kernel(z, edge_index):
    n_nodes, d_feat = z.shape
    n_edges = edge_index.shape[1]
    fn = _build(n_nodes, d_feat, n_edges)
    ei = edge_index.astype(jnp.int32)
    return fn(z, ei[0], ei[1])


# speedup vs baseline: 7.3276x; 2.8738x over previous
import functools

import jax
import jax.numpy as jnp
from jax import lax
from jax.experimental import pallas as pl
from jax.experimental.pallas import tpu as pltpu
from jax.experimental.pallas import tpu_sc as plsc

_NC = 2
_NS = 16
_NW = _NC * _NS
_CHUNK = 80

_DNUMS = lax.GatherDimensionNumbers(
    offset_dims=(), collapsed_slice_dims=(0,), start_index_map=(0,))


def _take16(x, idx):
    return lax.gather(x, idx[:, None], _DNUMS, (1,),
                      mode=lax.GatherScatterMode.PROMISE_IN_BOUNDS)


def _build(n_nodes, d_feat, n_edges):
    assert n_edges % _NW == 0
    edges_per_w = n_edges // _NW
    assert edges_per_w % _CHUNK == 0
    n_chunks = edges_per_w // _CHUNK
    n_grp = _CHUNK // 16
    n_k = d_feat // 16
    n_pairs = (n_chunks - 1) // 2

    mesh = plsc.VectorSubcoreMesh(
        core_axis_name="c", subcore_axis_name="s",
        num_cores=_NC, num_subcores=_NS)

    @functools.partial(
        pl.kernel,
        out_type=jax.ShapeDtypeStruct((n_edges,), jnp.float32),
        mesh=mesh,
        scratch_types=[
            pltpu.VMEM((edges_per_w,), jnp.int32),
            pltpu.VMEM((edges_per_w,), jnp.int32),
            pltpu.VMEM((2, _CHUNK, d_feat), jnp.float32),
            pltpu.VMEM((2, _CHUNK, d_feat), jnp.float32),
            pltpu.VMEM((edges_per_w,), jnp.float32),
            pltpu.SemaphoreType.DMA,
            pltpu.SemaphoreType.DMA,
            pltpu.SemaphoreType.DMA,
            pltpu.SemaphoreType.DMA,
        ],
    )
    def decoder(z_hbm, src_hbm, dst_hbm, out_hbm, si_v, di_v, sr_v, dr_v,
                o_v, sem_s0, sem_d0, sem_s1, sem_d1):
        wid = lax.axis_index("s") * _NC + lax.axis_index("c")
        base_w = wid * edges_per_w
        lane = lax.iota(jnp.int32, 16)
        masks = [(lane & d) == 0 for d in (1, 2, 4, 8)]
        perms = [lane ^ d for d in (1, 2, 4, 8)]
        sems = ((sem_s0, sem_d0), (sem_s1, sem_d1))

        pltpu.sync_copy(src_hbm.at[pl.ds(base_w, edges_per_w)], si_v)
        pltpu.sync_copy(dst_hbm.at[pl.ds(base_w, edges_per_w)], di_v)

        def fire(c, slot):
            s, d = sems[slot]
            pltpu.async_copy(
                z_hbm.at[si_v.at[pl.ds(c * _CHUNK, _CHUNK)]],
                sr_v.at[slot], s)
            pltpu.async_copy(
                z_hbm.at[di_v.at[pl.ds(c * _CHUNK, _CHUNK)]],
                dr_v.at[slot], d)

        def drain(slot):
            s, d = sems[slot]
            pltpu.make_async_copy(z_hbm.at[pl.ds(0, _CHUNK)],
                                  sr_v.at[slot], s).wait()
            pltpu.make_async_copy(z_hbm.at[pl.ds(0, _CHUNK)],
                                  dr_v.at[slot], d).wait()

        def compute(c, slot):
            sr, dr = sr_v.at[slot], dr_v.at[slot]

            @pl.loop(0, n_grp)
            def _grp(g):
                vecs = []
                for e in range(16):
                    r = g * 16 + e
                    acc = sr[r, pl.ds(0, 16)] * dr[r, pl.ds(0, 16)]
                    for k in range(1, n_k):
                        acc = acc + (sr[r, pl.ds(k * 16, 16)]
                                     * dr[r, pl.ds(k * 16, 16)])
                    vecs.append(acc)
                for m, p in zip(masks, perms):
                    vecs = [jnp.where(m, a, _take16(b, p))
                            + jnp.where(m, _take16(a, p), b)
                            for a, b in zip(vecs[0::2], vecs[1::2])]
                res = vecs[0]
                o_v[pl.ds(c * _CHUNK + g * 16, 16)] = 1.0 / (1.0 + jnp.exp(-res))

        fire(0, 0)

        @pl.loop(0, n_pairs)
        def _pair(j):
            c0 = 2 * j
            drain(0)
            fire(c0 + 1, 1)
            compute(c0, 0)
            drain(1)
            fire(c0 + 2, 0)
            compute(c0 + 1, 1)

        drain(0)
        compute(n_chunks - 1, 0)

        pltpu.sync_copy(o_v, out_hbm.at[pl.ds(base_w, edges_per_w)])

    return decoder


def kernel(z, edge_index):
    n_nodes, d_feat = z.shape
    n_edges = edge_index.shape[1]
    fn = _build(n_nodes, d_feat, n_edges)
    ei = edge_index.astype(jnp.int32)
    return fn(z, ei[0], ei[1])
